# baseline (speedup 1.0000x reference)
import functools

import jax
import jax.numpy as jnp
from jax import lax
from jax.experimental import pallas as pl
from jax.experimental.pallas import tpu as pltpu
from jax.experimental.pallas import tpu_sc as plsc

BATCH = 16384
EMB = 32
NC = 2
NS = 16
NW = NC * NS
BPW = BATCH // NW
CHUNK = 128
NCHUNK = BPW // CHUNK

_MESH = plsc.VectorSubcoreMesh(core_axis_name="c", subcore_axis_name="s")


@functools.partial(
    pl.kernel,
    out_type=(
        jax.ShapeDtypeStruct((BATCH, EMB), jnp.float32),
        jax.ShapeDtypeStruct((BATCH, EMB), jnp.float32),
    ),
    mesh=_MESH,
    compiler_params=pltpu.CompilerParams(use_tc_tiling_on_sc=False),
    scratch_types=[
        pltpu.VMEM((NCHUNK, CHUNK), jnp.int32),
        pltpu.VMEM((NCHUNK, CHUNK), jnp.int32),
        pltpu.VMEM((BPW, EMB), jnp.float32),
        pltpu.VMEM((BPW, EMB), jnp.float32),
        pltpu.SemaphoreType.DMA,
    ],
)
def _sc_gather(uid_hbm, vid_hbm, utab_hbm, vtab_hbm, ue_hbm, ve_hbm,
               uidx_v, vidx_v, urows_v, vrows_v, sem):
    wid = lax.axis_index("s") * NC + lax.axis_index("c")
    base = wid * BPW
    pltpu.sync_copy(uid_hbm.at[wid], uidx_v)
    pltpu.sync_copy(vid_hbm.at[wid], vidx_v)
    copies = []
    for j in range(NCHUNK):
        copies.append(pltpu.async_copy(
            utab_hbm.at[uidx_v.at[j]], urows_v.at[pl.ds(j * CHUNK, CHUNK)], sem))
        copies.append(pltpu.async_copy(
            vtab_hbm.at[vidx_v.at[j]], vrows_v.at[pl.ds(j * CHUNK, CHUNK)], sem))
    for c in copies:
        c.wait()
    pltpu.sync_copy(urows_v, ue_hbm.at[pl.ds(base, BPW)])
    pltpu.sync_copy(vrows_v, ve_hbm.at[pl.ds(base, BPW)])


_ROWS = 2048


def _mlp_body(ue, ve, w1a, w1b, b1, wo1, bo1, wo2, bo2, l1, l2):
    h = jnp.dot(ue[...], w1a[...], preferred_element_type=jnp.float32)
    h += jnp.dot(ve[...], w1b[...], preferred_element_type=jnp.float32)
    h = jnp.maximum(h + b1[...], 0.0)
    l1[...] = jnp.dot(h, wo1[...], preferred_element_type=jnp.float32) + bo1[...]
    l2[...] = jnp.dot(h, wo2[...], preferred_element_type=jnp.float32) + bo2[...]


def _mlp(ue, ve, w1a, w1b, b1, wo1, bo1, wo2, bo2):
    grid = (BATCH // _ROWS,)
    full = lambda shape: pl.BlockSpec(shape, lambda i: (0, 0))
    return pl.pallas_call(
        _mlp_body,
        grid=grid,
        in_specs=[
            pl.BlockSpec((_ROWS, EMB), lambda i: (i, 0)),
            pl.BlockSpec((_ROWS, EMB), lambda i: (i, 0)),
            full((EMB, 32)),
            full((EMB, 32)),
            full((1, 32)),
            full((32, 10)),
            full((1, 10)),
            full((32, 1)),
            full((1, 1)),
        ],
        out_specs=[
            pl.BlockSpec((_ROWS, 10), lambda i: (i, 0)),
            pl.BlockSpec((_ROWS, 1), lambda i: (i, 0)),
        ],
        out_shape=[
            jax.ShapeDtypeStruct((BATCH, 10), jnp.float32),
            jax.ShapeDtypeStruct((BATCH, 1), jnp.float32),
        ],
    )(ue, ve, w1a, w1b, b1, wo1, bo1, wo2, bo2)


def kernel(user_id, video_id, user_table, video_table, W1, b1, Wo1, bo1, Wo2, bo2):
    uid = jnp.asarray(user_id, jnp.int32).reshape(NW, NCHUNK, CHUNK)
    vid = jnp.asarray(video_id, jnp.int32).reshape(NW, NCHUNK, CHUNK)
    ue, ve = _sc_gather(uid, vid, user_table, video_table)
    l1, l2 = _mlp(ue, ve, W1[:EMB], W1[EMB:], b1.reshape(1, 32),
                  Wo1, bo1.reshape(1, 10), Wo2, bo2.reshape(1, 1))
    return (l1, l2)
